# baseline (speedup 1.0000x reference)
import functools

import jax
import jax.numpy as jnp
from jax import lax
from jax.experimental import pallas as pl
from jax.experimental.pallas import tpu as pltpu
from jax.experimental.pallas import tpu_sc as plsc

NC = 2
NS = 16
NW = NC * NS
G = 128
NBUF = 5
ROW = 128
TCB = 1024


def _tc_format_body(in_ref, out_ref):
    out_ref[:, 0:in_ref.shape[0]] = in_ref[...].T


def _format_table(V: int, D: int):
    return pl.pallas_call(
        _tc_format_body,
        grid=(pl.cdiv(V, TCB),),
        in_specs=[pl.BlockSpec((D, TCB), lambda i: (0, i))],
        out_specs=pl.BlockSpec((TCB, ROW), lambda i: (i, 0)),
        out_shape=jax.ShapeDtypeStruct((V, ROW), jnp.float32),
    )


def _make_gather(n_g: int, D: int):
    assert n_g % NBUF == 0
    n_outer = n_g // NBUF
    mesh = plsc.VectorSubcoreMesh(core_axis_name="c", subcore_axis_name="s")

    @functools.partial(
        pl.kernel,
        mesh=mesh,
        out_type=jax.ShapeDtypeStruct((NW * n_g * G, D), jnp.float32),
        scratch_types=(
            [pltpu.VMEM((n_g * G,), jnp.int32)]
            + [pltpu.VMEM((G, ROW), jnp.float32) for _ in range(NBUF)]
            + [pltpu.SemaphoreType.DMA for _ in range(NBUF)]
        ),
        compiler_params=pltpu.CompilerParams(use_tc_tiling_on_sc=False),
    )
    def gather_kernel(keys_hbm, idx_hbm, out_hbm, idx_v, *bufs_and_sems):
        bufs = bufs_and_sems[:NBUF]
        gsems = bufs_and_sems[NBUF:]
        wid = lax.axis_index("s") * NC + lax.axis_index("c")
        pltpu.sync_copy(idx_hbm.at[pl.ds(wid * n_g * G, n_g * G)], idx_v)
        base = wid * n_g * G

        for b in range(NBUF):
            pltpu.async_copy(
                keys_hbm.at[idx_v.at[pl.ds(b * G, G)]], bufs[b], gsems[b]
            )

        def outer(o, carry):
            for b in range(NBUF):
                g = o * NBUF + b
                pltpu.make_async_copy(
                    keys_hbm.at[idx_v.at[pl.ds(g * G, G)]], bufs[b], gsems[b]
                ).wait()
                pltpu.sync_copy(
                    bufs[b].at[:, pl.ds(0, D)],
                    out_hbm.at[pl.ds(base + g * G, G)],
                )

                @pl.when(o < n_outer - 1)
                def _():
                    pltpu.async_copy(
                        keys_hbm.at[idx_v.at[pl.ds((g + NBUF) * G, G)]],
                        bufs[b],
                        gsems[b],
                    )

            return carry

        lax.fori_loop(0, n_outer, outer, 0)

    return gather_kernel


def kernel(uids, keys):
    B, H = uids.shape
    V, D = keys.shape
    T = B * H
    assert T % (NW * G) == 0
    n_g = T // (NW * G)
    keys_p = _format_table(V, D)(keys.T)
    idx = uids.reshape(T)
    out = _make_gather(n_g, D)(keys_p, idx)
    return out.reshape(B, H, D)


# speedup vs baseline: 1.3256x; 1.3256x over previous
import functools

import jax
import jax.numpy as jnp
from jax import lax
from jax.experimental import pallas as pl
from jax.experimental.pallas import tpu as pltpu
from jax.experimental.pallas import tpu_sc as plsc

NC = 2
NS = 16
NW = NC * NS
G = 128
NBUF = 5
ROW = 128
TCB = 2048


def _tc_format_body(in_ref, out_ref):
    d = in_ref.shape[0]
    iden = jnp.eye(d, dtype=jnp.float32)
    out_ref[:, 0:d] = jax.lax.dot_general(
        in_ref[...],
        iden,
        (((0,), (0,)), ((), ())),
        preferred_element_type=jnp.float32,
    )


def _format_table(V: int, D: int):
    return pl.pallas_call(
        _tc_format_body,
        grid=(pl.cdiv(V, TCB),),
        in_specs=[pl.BlockSpec((D, TCB), lambda i: (0, i))],
        out_specs=pl.BlockSpec((TCB, ROW), lambda i: (i, 0)),
        out_shape=jax.ShapeDtypeStruct((V, ROW), jnp.float32),
    )


def _make_gather(n_g: int, D: int):
    assert n_g % NBUF == 0
    n_outer = n_g // NBUF
    mesh = plsc.VectorSubcoreMesh(core_axis_name="c", subcore_axis_name="s")

    @functools.partial(
        pl.kernel,
        mesh=mesh,
        out_type=jax.ShapeDtypeStruct((NW * n_g * G, D), jnp.float32),
        scratch_types=(
            [pltpu.VMEM((n_g * G,), jnp.int32)]
            + [pltpu.VMEM((G, ROW), jnp.float32) for _ in range(NBUF)]
            + [pltpu.SemaphoreType.DMA for _ in range(NBUF)]
        ),
        compiler_params=pltpu.CompilerParams(use_tc_tiling_on_sc=False),
    )
    def gather_kernel(keys_hbm, idx_hbm, out_hbm, idx_v, *bufs_and_sems):
        bufs = bufs_and_sems[:NBUF]
        gsems = bufs_and_sems[NBUF:]
        wid = lax.axis_index("s") * NC + lax.axis_index("c")
        pltpu.sync_copy(idx_hbm.at[pl.ds(wid * n_g * G, n_g * G)], idx_v)
        base = wid * n_g * G

        for b in range(NBUF):
            pltpu.async_copy(
                keys_hbm.at[idx_v.at[pl.ds(b * G, G)]], bufs[b], gsems[b]
            )

        def outer(o, carry):
            for b in range(NBUF):
                g = o * NBUF + b
                pltpu.make_async_copy(
                    keys_hbm.at[idx_v.at[pl.ds(g * G, G)]], bufs[b], gsems[b]
                ).wait()
                pltpu.sync_copy(
                    bufs[b].at[:, pl.ds(0, D)],
                    out_hbm.at[pl.ds(base + g * G, G)],
                )

                @pl.when(o < n_outer - 1)
                def _():
                    pltpu.async_copy(
                        keys_hbm.at[idx_v.at[pl.ds((g + NBUF) * G, G)]],
                        bufs[b],
                        gsems[b],
                    )

            return carry

        lax.fori_loop(0, n_outer, outer, 0)

    return gather_kernel


def kernel(uids, keys):
    B, H = uids.shape
    V, D = keys.shape
    T = B * H
    assert T % (NW * G) == 0
    n_g = T // (NW * G)
    keys_p = _format_table(V, D)(keys.T)
    idx = uids.reshape(T)
    out = _make_gather(n_g, D)(keys_p, idx)
    return out.reshape(B, H, D)
